# baseline (speedup 1.0000x reference)
import jax
import jax.numpy as jnp
from jax.experimental import pallas as pl
from jax.experimental.pallas import tpu as pltpu

_H = 512
_W = 512
_T = 8
_IPB = 8


def _erode(xs, prev_row):
    xu = jnp.concatenate([prev_row, xs[:-1]], axis=0)
    er = jnp.minimum(xs, xu)
    el = jnp.concatenate([er[:, :1], er[:, :-1]], axis=1)
    return jnp.minimum(er, el)


def _dilate_sqerr(e, next_row, x):
    ed = jnp.concatenate([e[1:], next_row], axis=0)
    d = jnp.maximum(e, ed)
    dr = jnp.concatenate([d[:, 1:], d[:, -1:]], axis=1)
    d = jnp.maximum(d, dr)
    diff = x - d
    return diff * diff


def _opening_mse_body(x_ref, out_ref):
    j = pl.program_id(1)
    n_tiles = _H // _T

    def img_body(k, acc):
        xs0 = x_ref[k, 0:_T, :]
        e0 = _erode(xs0, xs0[0:1])

        def tile_body(t, carry):
            x_prev, e_prev, a = carry
            r = pl.multiple_of(t * _T, _T)
            xs = x_ref[k, pl.ds(r, _T), :]
            e = _erode(xs, x_prev[_T - 1:_T])
            a = a + _dilate_sqerr(e_prev, e[0:1], x_prev)
            return (xs, e, a)

        x_l, e_l, acc = jax.lax.fori_loop(
            1, n_tiles, tile_body, (xs0, e0, acc))
        return acc + _dilate_sqerr(e_l, e_l[_T - 1:_T], x_l)

    acc = jax.lax.fori_loop(
        0, _IPB, img_body, jnp.zeros((_T, _W), jnp.float32))
    total = jnp.sum(acc).reshape(1, 1, 1)

    @pl.when(j == 0)
    def _():
        out_ref[...] = total

    @pl.when(j != 0)
    def _():
        out_ref[...] = out_ref[...] + total


def kernel(labels):
    b, c, h, w = labels.shape
    n = b * c
    x = labels.reshape(n, h, w)
    per_core = n // 2 // _IPB
    partials = pl.pallas_call(
        _opening_mse_body,
        grid=(2, per_core),
        in_specs=[pl.BlockSpec((_IPB, h, w),
                               lambda i, j: (i * per_core + j, 0, 0))],
        out_specs=pl.BlockSpec((1, 1, 1), lambda i, j: (i, 0, 0)),
        out_shape=jax.ShapeDtypeStruct((2, 1, 1), jnp.float32),
        compiler_params=pltpu.CompilerParams(
            dimension_semantics=("parallel", "arbitrary"),
        ),
    )(x)
    return jnp.sum(partials) / (n * h * w)


# speedup vs baseline: 8.1341x; 8.1341x over previous
import jax
import jax.numpy as jnp
from jax.experimental import pallas as pl
from jax.experimental.pallas import tpu as pltpu

_H = 512
_W = 512
_STRIP = 32
_IPB = 8


def _erode(xs, prev_row):
    xu = jnp.concatenate([prev_row, xs[:-1]], axis=0)
    er = jnp.minimum(xs, xu)
    el = jnp.concatenate([er[:, :1], er[:, :-1]], axis=1)
    return jnp.minimum(er, el)


def _dilate_sqerr(e, next_row, x):
    ed = jnp.concatenate([e[1:], next_row], axis=0)
    d = jnp.maximum(e, ed)
    dr = jnp.concatenate([d[:, 1:], d[:, -1:]], axis=1)
    d = jnp.maximum(d, dr)
    diff = x - d
    return diff * diff


def _fold(acc, d2):
    for m in range(d2.shape[0] // 8):
        acc = acc + d2[8 * m:8 * m + 8]
    return acc


def _opening_mse_body(x_ref, out_ref):
    j = pl.program_id(1)
    n_strips = _H // _STRIP

    def img_body(k, acc):
        x_prev = None
        e_prev = None
        for s in range(n_strips):
            xs = x_ref[k, s * _STRIP:(s + 1) * _STRIP, :]
            if s == 0:
                prev_row = xs[0:1]
            else:
                prev_row = x_prev[_STRIP - 1:_STRIP]
            e = _erode(xs, prev_row)
            if s > 0:
                acc = _fold(acc, _dilate_sqerr(e_prev, e[0:1], x_prev))
            x_prev, e_prev = xs, e
        return _fold(acc, _dilate_sqerr(
            e_prev, e_prev[_STRIP - 1:_STRIP], x_prev))

    acc = jax.lax.fori_loop(
        0, _IPB, img_body, jnp.zeros((8, _W), jnp.float32))
    total = jnp.sum(acc).reshape(1, 1, 1)

    @pl.when(j == 0)
    def _():
        out_ref[...] = total

    @pl.when(j != 0)
    def _():
        out_ref[...] = out_ref[...] + total


def kernel(labels):
    b, c, h, w = labels.shape
    n = b * c
    x = labels.reshape(n, h, w)
    per_core = n // 2 // _IPB
    partials = pl.pallas_call(
        _opening_mse_body,
        grid=(2, per_core),
        in_specs=[pl.BlockSpec((_IPB, h, w),
                               lambda i, j: (i * per_core + j, 0, 0))],
        out_specs=pl.BlockSpec((1, 1, 1), lambda i, j: (i, 0, 0)),
        out_shape=jax.ShapeDtypeStruct((2, 1, 1), jnp.float32),
        compiler_params=pltpu.CompilerParams(
            dimension_semantics=("parallel", "arbitrary"),
        ),
    )(x)
    return jnp.sum(partials) / (n * h * w)


# speedup vs baseline: 12.5196x; 1.5391x over previous
import jax
import jax.numpy as jnp
from jax.experimental import pallas as pl
from jax.experimental.pallas import tpu as pltpu

_H = 512
_W = 512
_STRIP = 16
_IPB = 8


def _erode_pair(xs, prev_row):
    xu = jnp.concatenate([prev_row, xs[:-1]], axis=0)
    r = jnp.minimum(xs, xu)
    rm = jnp.concatenate([r[:, :1], r[:, :-1]], axis=1)
    rp = jnp.concatenate([r[:, 1:], r[:, _W - 2:_W - 1]], axis=1)
    return jnp.minimum(r, rm), jnp.minimum(r, rp)


def _dilate_sqerr(e, e_next, e_row, en_row, x):
    ed = jnp.concatenate([e[1:], e_row], axis=0)
    d = jnp.maximum(e, ed)
    end_ = jnp.concatenate([e_next[1:], en_row], axis=0)
    dn = jnp.maximum(e_next, end_)
    opened = jnp.maximum(d, dn)
    diff = x - opened
    return diff * diff


def _fold(acc, d2):
    for m in range(d2.shape[0] // 8):
        acc = acc + d2[8 * m:8 * m + 8]
    return acc


def _opening_mse_body(x_ref, out_ref):
    j = pl.program_id(1)
    n_strips = _H // _STRIP

    def img_body(k, acc):
        x_prev = e_prev = en_prev = None
        for s in range(n_strips):
            xs = x_ref[k, s * _STRIP:(s + 1) * _STRIP, :]
            if s == 0:
                prev_row = xs[0:1]
            else:
                prev_row = x_prev[_STRIP - 1:_STRIP]
            e, en = _erode_pair(xs, prev_row)
            if s > 0:
                acc = _fold(acc, _dilate_sqerr(
                    e_prev, en_prev, e[0:1], en[0:1], x_prev))
            x_prev, e_prev, en_prev = xs, e, en
        last = _STRIP - 1
        return _fold(acc, _dilate_sqerr(
            e_prev, en_prev, e_prev[last:last + 1],
            en_prev[last:last + 1], x_prev))

    acc = jax.lax.fori_loop(
        0, _IPB, img_body, jnp.zeros((8, _W), jnp.float32))
    total = jnp.sum(acc).reshape(1, 1, 1)

    @pl.when(j == 0)
    def _():
        out_ref[...] = total

    @pl.when(j != 0)
    def _():
        out_ref[...] = out_ref[...] + total


def kernel(labels):
    b, c, h, w = labels.shape
    n = b * c
    x = labels.reshape(n, h, w)
    per_core = n // 2 // _IPB
    partials = pl.pallas_call(
        _opening_mse_body,
        grid=(2, per_core),
        in_specs=[pl.BlockSpec((_IPB, h, w),
                               lambda i, j: (i * per_core + j, 0, 0))],
        out_specs=pl.BlockSpec((1, 1, 1), lambda i, j: (i, 0, 0)),
        out_shape=jax.ShapeDtypeStruct((2, 1, 1), jnp.float32),
        compiler_params=pltpu.CompilerParams(
            dimension_semantics=("parallel", "arbitrary"),
        ),
    )(x)
    return jnp.sum(partials) / (n * h * w)


# speedup vs baseline: 13.4151x; 1.0715x over previous
import jax
import jax.numpy as jnp
from jax.experimental import pallas as pl
from jax.experimental.pallas import tpu as pltpu

_H = 512
_W = 512
_STRIP = 16
_IPB = 8


def _erode_pair(xs, prev_row):
    xu = jnp.concatenate([prev_row, xs[:-1]], axis=0)
    r = jnp.minimum(xs, xu)
    rm = jnp.concatenate([r[:, :1], r[:, :-1]], axis=1)
    rp = jnp.concatenate([r[:, 1:], r[:, _W - 2:_W - 1]], axis=1)
    return jnp.minimum(r, rm), jnp.minimum(r, rp)


def _dilate_sqerr(g, g_row, x):
    gd = jnp.concatenate([g[1:], g_row], axis=0)
    opened = jnp.maximum(g, gd)
    diff = x - opened
    return diff * diff


def _fold(acc, d2):
    for m in range(d2.shape[0] // 8):
        acc = acc + d2[8 * m:8 * m + 8]
    return acc


def _opening_mse_body(x_ref, out_ref):
    j = pl.program_id(1)
    n_strips = _H // _STRIP

    def img_body(k, acc):
        x_prev = g_prev = None
        for s in range(n_strips):
            xs = x_ref[k, s * _STRIP:(s + 1) * _STRIP, :]
            if s == 0:
                prev_row = xs[0:1]
            else:
                prev_row = x_prev[_STRIP - 1:_STRIP]
            e, en = _erode_pair(xs, prev_row)
            g = jnp.maximum(e, en)
            if s > 0:
                acc = _fold(acc, _dilate_sqerr(g_prev, g[0:1], x_prev))
            x_prev, g_prev = xs, g
        last = _STRIP - 1
        return _fold(acc, _dilate_sqerr(
            g_prev, g_prev[last:last + 1], x_prev))

    acc = jax.lax.fori_loop(
        0, _IPB, img_body, jnp.zeros((8, _W), jnp.float32))
    total = jnp.sum(acc).reshape(1, 1, 1)

    @pl.when(j == 0)
    def _():
        out_ref[...] = total

    @pl.when(j != 0)
    def _():
        out_ref[...] = out_ref[...] + total


def kernel(labels):
    b, c, h, w = labels.shape
    n = b * c
    x = labels.reshape(n, h, w)
    per_core = n // 2 // _IPB
    partials = pl.pallas_call(
        _opening_mse_body,
        grid=(2, per_core),
        in_specs=[pl.BlockSpec((_IPB, h, w),
                               lambda i, j: (i * per_core + j, 0, 0))],
        out_specs=pl.BlockSpec((1, 1, 1), lambda i, j: (i, 0, 0)),
        out_shape=jax.ShapeDtypeStruct((2, 1, 1), jnp.float32),
        compiler_params=pltpu.CompilerParams(
            dimension_semantics=("parallel", "arbitrary"),
        ),
    )(x)
    return jnp.sum(partials) / (n * h * w)
